# baseline (speedup 1.0000x reference)
import functools

import jax
import jax.numpy as jnp
from jax import lax
from jax.experimental import pallas as pl
from jax.experimental.pallas import tpu as pltpu
from jax.experimental.pallas import tpu_sc as plsc

B = 16
N = 20000
NP = 20480
T = 30
EVAL_NUM = 6
NMS_THRESHOLD = 2.0
_BIG_I = 2 ** 30
_EPS = 1e-6


def _smooth_l1_elt(d):
    ad = jnp.abs(d)
    return jnp.where(ad < 1.0, 0.5 * d * d, ad - 0.5)


def _tc_body(cx_ref, cy_ref, cls_ref, cen_ref, tgt_ref, f_ref, i_ref):
    cx = cx_ref[:]
    cy = cy_ref[:]
    cls = cls_ref[:]
    cen = cen_ref[:]
    tx = tgt_ref[:, 0:1]
    ty = tgt_ref[:, 1:2]
    lanes = lax.broadcasted_iota(jnp.int32, (B, NP), 1)
    valid = lanes < N

    dx = cx - tx
    dy = cy - ty
    dist = jnp.sqrt(dx * dx + dy * dy + 1e-12)
    dist = jnp.where(valid, dist, jnp.inf)
    sc = jnp.where(valid, cls * cen, -jnp.inf)

    d_cur = dist
    cls_sum = jnp.zeros((B, 1), jnp.float32)
    idx0 = None
    pick0 = None
    dist0 = None
    for k in range(EVAL_NUM):
        m = jnp.min(d_cur, axis=1, keepdims=True)
        j = jnp.min(jnp.where(d_cur == m, lanes, _BIG_I), axis=1, keepdims=True)
        pick = lanes == j
        p = jnp.sum(jnp.where(pick, cls, 0.0), axis=1, keepdims=True)
        p = jnp.clip(p, _EPS, 1.0 - _EPS)
        cls_sum = cls_sum - jnp.log(p)
        if k == 0:
            idx0, pick0, dist0 = j, pick, m
        d_cur = jnp.where(pick, jnp.inf, d_cur)
    class_loss = cls_sum / EVAL_NUM

    px = jnp.sum(jnp.where(pick0, cx, 0.0), axis=1, keepdims=True)
    py = jnp.sum(jnp.where(pick0, cy, 0.0), axis=1, keepdims=True)
    point_loss = 0.5 * (_smooth_l1_elt(px - tx) + _smooth_l1_elt(py - ty))
    cen0 = jnp.sum(jnp.where(pick0, cen, 0.0), axis=1, keepdims=True)
    cgt = jnp.where(dist0 >= 2.0, 0.0, 1.0 - jnp.sqrt(dist0 / 2.0))
    pc = jnp.clip(cen0, _EPS, 1.0 - _EPS)
    centerness_loss = -(cgt * jnp.log(pc) + (1.0 - cgt) * jnp.log(1.0 - pc))
    part_loss = class_loss + point_loss + centerness_loss

    ms = jnp.max(sc, axis=1, keepdims=True)
    bj = jnp.min(jnp.where(sc == ms, lanes, _BIG_I), axis=1, keepdims=True)
    de = jnp.sum(jnp.where(lanes == bj, dist, 0.0), axis=1, keepdims=True)

    sc_cur = sc
    probs = []
    gxs = []
    gys = []
    kidx = []
    for _ in range(EVAL_NUM):
        m = jnp.max(sc_cur, axis=1, keepdims=True)
        j = jnp.min(jnp.where(sc_cur == m, lanes, _BIG_I), axis=1, keepdims=True)
        pick = lanes == j
        cxj = jnp.sum(jnp.where(pick, cx, 0.0), axis=1, keepdims=True)
        cyj = jnp.sum(jnp.where(pick, cy, 0.0), axis=1, keepdims=True)
        probs.append(m)
        gxs.append(cxj)
        gys.append(cyj)
        kidx.append(j)
        ddx = cx - cxj
        ddy = cy - cyj
        dd = jnp.sqrt(ddx * ddx + ddy * ddy + 1e-12)
        sc_cur = jnp.where(dd < NMS_THRESHOLD, -jnp.inf, sc_cur)

    zero = jnp.zeros((B, 1), jnp.float32)
    f_ref[:] = jnp.concatenate(
        [part_loss, de] + probs + gxs + gys + [zero, zero, zero, zero], axis=1)

    brow = lax.broadcasted_iota(jnp.int32, (B, 1), 0) * N
    izero = jnp.zeros((B, 1), jnp.int32)
    i_ref[:] = jnp.concatenate(
        [idx0 + brow] + [j + brow for j in kidx] + [izero] * 9, axis=1)


@functools.cache
def _make_sc_gather():
    mesh = plsc.VectorSubcoreMesh(core_axis_name="c", subcore_axis_name="s")

    @functools.partial(
        pl.kernel,
        mesh=mesh,
        out_type=[
            jax.ShapeDtypeStruct((B, 8, 2 * T), jnp.float32),
            jax.ShapeDtypeStruct((B, 16), jnp.float32),
        ],
        scratch_types=[
            pltpu.VMEM((16,), jnp.int32),
            pltpu.VMEM((8, 2 * T), jnp.float32),
            pltpu.VMEM((64,), jnp.float32),
            pltpu.VMEM((16,), jnp.float32),
            pltpu.SemaphoreType.DMA,
        ],
        compiler_params=pltpu.CompilerParams(use_tc_tiling_on_sc=False),
    )
    def _sc_gather(traj_hbm, gidx_hbm, gt_hbm, rows_hbm, tl_hbm,
                   idx_v, rows_v, gt_v, tl_v, sem):
        c = lax.axis_index("c")
        s = lax.axis_index("s")
        wid = s * 2 + c

        @pl.when(wid < B)
        def _():
            b = wid
            pltpu.sync_copy(gidx_hbm.at[b], idx_v)
            pltpu.sync_copy(gt_hbm.at[b], gt_v)
            ivec = idx_v[...]
            copies = [
                pltpu.async_copy(traj_hbm.at[ivec[i]], rows_v.at[i], sem)
                for i in range(7)
            ]
            for cp in copies:
                cp.wait()
            pltpu.sync_copy(rows_v, rows_hbm.at[b])

            lane = lax.broadcasted_iota(jnp.int32, (16,), 0)
            acc = jnp.zeros((16,), jnp.float32)
            for off, mstart in ((0, 0), (16, 0), (32, 0), (44, 4)):
                d = rows_v[0, pl.ds(off, 16)] - gt_v[pl.ds(off, 16)]
                v = _smooth_l1_elt(d)
                if mstart:
                    v = jnp.where(lane >= mstart, v, 0.0)
                acc = acc + v
            total = acc[0]
            for i in range(1, 16):
                total = total + acc[i]
            tl_v[:] = jnp.zeros((16,), jnp.float32) + total * (1.0 / (2.0 * T))
            pltpu.sync_copy(tl_v, tl_hbm.at[b])

    return _sc_gather


def kernel(outputs_coord, outputs_class, outputs_traj, outputs_centerness,
           gt_points):
    cx = jnp.pad(outputs_coord[..., 0], ((0, 0), (0, NP - N)))
    cy = jnp.pad(outputs_coord[..., 1], ((0, 0), (0, NP - N)))
    cls = jnp.pad(outputs_class, ((0, 0), (0, NP - N)))
    cen = jnp.pad(outputs_centerness, ((0, 0), (0, NP - N)))
    tgt = gt_points[:, -1, :]

    f_out, i_out = pl.pallas_call(
        _tc_body,
        out_shape=[
            jax.ShapeDtypeStruct((B, 24), jnp.float32),
            jax.ShapeDtypeStruct((B, 16), jnp.int32),
        ],
    )(cx, cy, cls, cen, tgt)

    traj_flat = outputs_traj.reshape(B * N, 2 * T)
    gt_flat = jnp.pad(gt_points.reshape(B, 2 * T), ((0, 0), (0, 4)))
    rows, tl = _make_sc_gather()(traj_flat, i_out, gt_flat)

    total_loss = f_out[:, 0] + tl[:, 0]
    de = f_out[:, 1]
    pred_probs = f_out[:, 2:8]
    pred_goals = jnp.stack([f_out[:, 8:14], f_out[:, 14:20]], axis=-1)
    pred_trajs = rows[:, 1:7, :].reshape(B, EVAL_NUM, T, 2)
    return (total_loss, de, pred_goals, pred_probs, pred_trajs)


# speedup vs baseline: 6.4711x; 6.4711x over previous
import functools

import jax
import jax.numpy as jnp
from jax import lax
from jax.experimental import pallas as pl
from jax.experimental.pallas import tpu as pltpu
from jax.experimental.pallas import tpu_sc as plsc

B = 16
N = 20000
NP = 20480
T = 30
EVAL_NUM = 6
NMS_THRESHOLD = 2.0
_BIG_I = 2 ** 30
_EPS = 1e-6


def _smooth_l1_elt(d):
    ad = jnp.abs(d)
    return jnp.where(ad < 1.0, 0.5 * d * d, ad - 0.5)


def _tc_body(cx_ref, cy_ref, cls_ref, cen_ref, tgt_ref, f_ref, i_ref):
    cx = cx_ref[:]
    cy = cy_ref[:]
    cls = cls_ref[:]
    cen = cen_ref[:]
    tx = tgt_ref[:, 0:1]
    ty = tgt_ref[:, 1:2]
    lanes = lax.broadcasted_iota(jnp.int32, (B, NP), 1)
    valid = lanes < N

    dx = cx - tx
    dy = cy - ty
    dist = jnp.sqrt(dx * dx + dy * dy + 1e-12)
    dist = jnp.where(valid, dist, jnp.inf)
    sc = jnp.where(valid, cls * cen, -jnp.inf)

    d_cur = dist
    cls_sum = jnp.zeros((B, 1), jnp.float32)
    idx0 = None
    pick0 = None
    dist0 = None
    for k in range(EVAL_NUM):
        m = jnp.min(d_cur, axis=1, keepdims=True)
        j = jnp.min(jnp.where(d_cur == m, lanes, _BIG_I), axis=1, keepdims=True)
        pick = lanes == j
        p = jnp.sum(jnp.where(pick, cls, 0.0), axis=1, keepdims=True)
        p = jnp.clip(p, _EPS, 1.0 - _EPS)
        cls_sum = cls_sum - jnp.log(p)
        if k == 0:
            idx0, pick0, dist0 = j, pick, m
        d_cur = jnp.where(pick, jnp.inf, d_cur)
    class_loss = cls_sum / EVAL_NUM

    px = jnp.sum(jnp.where(pick0, cx, 0.0), axis=1, keepdims=True)
    py = jnp.sum(jnp.where(pick0, cy, 0.0), axis=1, keepdims=True)
    point_loss = 0.5 * (_smooth_l1_elt(px - tx) + _smooth_l1_elt(py - ty))
    cen0 = jnp.sum(jnp.where(pick0, cen, 0.0), axis=1, keepdims=True)
    cgt = jnp.where(dist0 >= 2.0, 0.0, 1.0 - jnp.sqrt(dist0 / 2.0))
    pc = jnp.clip(cen0, _EPS, 1.0 - _EPS)
    centerness_loss = -(cgt * jnp.log(pc) + (1.0 - cgt) * jnp.log(1.0 - pc))
    part_loss = class_loss + point_loss + centerness_loss

    ms = jnp.max(sc, axis=1, keepdims=True)
    bj = jnp.min(jnp.where(sc == ms, lanes, _BIG_I), axis=1, keepdims=True)
    de = jnp.sum(jnp.where(lanes == bj, dist, 0.0), axis=1, keepdims=True)

    sc_cur = sc
    probs = []
    gxs = []
    gys = []
    kidx = []
    for _ in range(EVAL_NUM):
        m = jnp.max(sc_cur, axis=1, keepdims=True)
        j = jnp.min(jnp.where(sc_cur == m, lanes, _BIG_I), axis=1, keepdims=True)
        pick = lanes == j
        cxj = jnp.sum(jnp.where(pick, cx, 0.0), axis=1, keepdims=True)
        cyj = jnp.sum(jnp.where(pick, cy, 0.0), axis=1, keepdims=True)
        probs.append(m)
        gxs.append(cxj)
        gys.append(cyj)
        kidx.append(j)
        ddx = cx - cxj
        ddy = cy - cyj
        dd = jnp.sqrt(ddx * ddx + ddy * ddy + 1e-12)
        sc_cur = jnp.where(dd < NMS_THRESHOLD, -jnp.inf, sc_cur)

    zero = jnp.zeros((B, 1), jnp.float32)
    f_ref[:] = jnp.concatenate(
        [part_loss, de] + probs + gxs + gys + [zero, zero, zero, zero], axis=1)

    izero = jnp.zeros((B, 1), jnp.int32)
    i_ref[:] = jnp.concatenate([idx0] + kidx + [izero] * 9, axis=1)


def _gather_body(idx_ref, traj_ref, gt_ref, rows_ref, tl_ref):
    b = pl.program_id(0)
    s = pl.program_id(1)
    idx = idx_ref[b, s]
    off = lax.rem(idx, 128)
    lane = lax.broadcasted_iota(jnp.int32, (1, T, 2, 128), 3)
    blk = traj_ref[...]
    row = jnp.sum(jnp.where(lane == off, blk, 0.0), axis=3)
    rows_ref[...] = row[:, None]

    d = row - gt_ref[...]
    sl = jnp.sum(_smooth_l1_elt(d)) * (1.0 / (2 * T))
    tl_ref[...] = jnp.zeros((1, 1, 1, 1), jnp.float32) + jnp.where(s == 0, sl, 0.0)


def kernel(outputs_coord, outputs_class, outputs_traj, outputs_centerness,
           gt_points):
    cx = jnp.pad(outputs_coord[..., 0], ((0, 0), (0, NP - N)))
    cy = jnp.pad(outputs_coord[..., 1], ((0, 0), (0, NP - N)))
    cls = jnp.pad(outputs_class, ((0, 0), (0, NP - N)))
    cen = jnp.pad(outputs_centerness, ((0, 0), (0, NP - N)))
    tgt = gt_points[:, -1, :]

    f_out, i_out = pl.pallas_call(
        _tc_body,
        out_shape=[
            jax.ShapeDtypeStruct((B, 24), jnp.float32),
            jax.ShapeDtypeStruct((B, 16), jnp.int32),
        ],
    )(cx, cy, cls, cen, tgt)

    traj_v = jnp.transpose(outputs_traj, (0, 2, 3, 1))

    rows, tl7 = pl.pallas_call(
        _gather_body,
        grid_spec=pltpu.PrefetchScalarGridSpec(
            num_scalar_prefetch=1,
            grid=(B, 7),
            in_specs=[
                pl.BlockSpec((1, T, 2, 128),
                             lambda b, s, idx_ref: (b, 0, 0, idx_ref[b, s] // 128)),
                pl.BlockSpec((1, T, 2), lambda b, s, idx_ref: (b, 0, 0)),
            ],
            out_specs=[
                pl.BlockSpec((1, 1, T, 2), lambda b, s, idx_ref: (b, s, 0, 0)),
                pl.BlockSpec((1, 1, 1, 1),
                             lambda b, s, idx_ref: (b, s, 0, 0)),
            ],
        ),
        out_shape=[
            jax.ShapeDtypeStruct((B, 7, T, 2), jnp.float32),
            jax.ShapeDtypeStruct((B, 7, 1, 1), jnp.float32),
        ],
    )(i_out, traj_v, gt_points)

    total_loss = f_out[:, 0] + tl7[:, 0, 0, 0]
    de = f_out[:, 1]
    pred_probs = f_out[:, 2:8]
    pred_goals = jnp.stack([f_out[:, 8:14], f_out[:, 14:20]], axis=-1)
    pred_trajs = rows[:, 1:7]
    return (total_loss, de, pred_goals, pred_probs, pred_trajs)


# speedup vs baseline: 15.6260x; 2.4147x over previous
import functools

import jax
import jax.numpy as jnp
from jax import lax
from jax.experimental import pallas as pl
from jax.experimental.pallas import tpu as pltpu
from jax.experimental.pallas import tpu_sc as plsc

B = 16
N = 20000
NP = 20480
T = 30
EVAL_NUM = 6
NMS_THRESHOLD = 2.0
_BIG_I = 2 ** 30
_EPS = 1e-6


def _smooth_l1_elt(d):
    ad = jnp.abs(d)
    return jnp.where(ad < 1.0, 0.5 * d * d, ad - 0.5)


def _tc_body(co_ref, cls_ref, cen_ref, tgt_ref, f_ref, i_ref):
    cx = co_ref[:, 0, :]
    cy = co_ref[:, 1, :]
    cls = cls_ref[:]
    cen = cen_ref[:]
    tx = tgt_ref[:, 0:1]
    ty = tgt_ref[:, 1:2]
    lanes = lax.broadcasted_iota(jnp.int32, (B, N), 1)

    dx = cx - tx
    dy = cy - ty
    dist = jnp.sqrt(dx * dx + dy * dy + 1e-12)
    sc = cls * cen

    d_cur = dist
    cls_sum = jnp.zeros((B, 1), jnp.float32)
    idx0 = None
    pick0 = None
    dist0 = None
    for k in range(EVAL_NUM):
        m = jnp.min(d_cur, axis=1, keepdims=True)
        j = jnp.min(jnp.where(d_cur == m, lanes, _BIG_I), axis=1, keepdims=True)
        pick = lanes == j
        p = jnp.sum(jnp.where(pick, cls, 0.0), axis=1, keepdims=True)
        p = jnp.clip(p, _EPS, 1.0 - _EPS)
        cls_sum = cls_sum - jnp.log(p)
        if k == 0:
            idx0, pick0, dist0 = j, pick, m
        d_cur = jnp.where(pick, jnp.inf, d_cur)
    class_loss = cls_sum / EVAL_NUM

    px = jnp.sum(jnp.where(pick0, cx, 0.0), axis=1, keepdims=True)
    py = jnp.sum(jnp.where(pick0, cy, 0.0), axis=1, keepdims=True)
    point_loss = 0.5 * (_smooth_l1_elt(px - tx) + _smooth_l1_elt(py - ty))
    cen0 = jnp.sum(jnp.where(pick0, cen, 0.0), axis=1, keepdims=True)
    cgt = jnp.where(dist0 >= 2.0, 0.0, 1.0 - jnp.sqrt(dist0 / 2.0))
    pc = jnp.clip(cen0, _EPS, 1.0 - _EPS)
    centerness_loss = -(cgt * jnp.log(pc) + (1.0 - cgt) * jnp.log(1.0 - pc))
    part_loss = class_loss + point_loss + centerness_loss

    ms = jnp.max(sc, axis=1, keepdims=True)
    bj = jnp.min(jnp.where(sc == ms, lanes, _BIG_I), axis=1, keepdims=True)
    de = jnp.sum(jnp.where(lanes == bj, dist, 0.0), axis=1, keepdims=True)

    sc_cur = sc
    probs = []
    gxs = []
    gys = []
    kidx = []
    for _ in range(EVAL_NUM):
        m = jnp.max(sc_cur, axis=1, keepdims=True)
        j = jnp.min(jnp.where(sc_cur == m, lanes, _BIG_I), axis=1, keepdims=True)
        pick = lanes == j
        cxj = jnp.sum(jnp.where(pick, cx, 0.0), axis=1, keepdims=True)
        cyj = jnp.sum(jnp.where(pick, cy, 0.0), axis=1, keepdims=True)
        probs.append(m)
        gxs.append(cxj)
        gys.append(cyj)
        kidx.append(j)
        ddx = cx - cxj
        ddy = cy - cyj
        dd = jnp.sqrt(ddx * ddx + ddy * ddy + 1e-12)
        sc_cur = jnp.where(dd < NMS_THRESHOLD, -jnp.inf, sc_cur)

    zero = jnp.zeros((B, 1), jnp.float32)
    f_ref[:] = jnp.concatenate(
        [part_loss, de] + probs + gxs + gys + [zero, zero, zero, zero], axis=1)

    izero = jnp.zeros((B, 1), jnp.int32)
    i_ref[:] = jnp.concatenate([idx0] + kidx + [izero] * 9, axis=1)


def _gather_body(idx_ref, *refs):
    traj_refs = refs[:7]
    gt_ref = refs[7]
    rows_ref = refs[8]
    tl_ref = refs[9]
    b = pl.program_id(0)
    lane = lax.broadcasted_iota(jnp.int32, (1, T, 2, 128), 3)
    for s in range(7):
        off = lax.rem(idx_ref[b, s], 128)
        blk = traj_refs[s][...]
        row = jnp.sum(jnp.where(lane == off, blk, 0.0), axis=3)
        rows_ref[0, s] = row[0]
        if s == 0:
            d = row - gt_ref[...]
            sl = jnp.sum(_smooth_l1_elt(d)) * (1.0 / (2 * T))
            tl_ref[...] = jnp.zeros((1, 1, 1, 1), jnp.float32) + sl


def kernel(outputs_coord, outputs_class, outputs_traj, outputs_centerness,
           gt_points):
    coord_t = jnp.transpose(outputs_coord, (0, 2, 1))
    tgt = gt_points[:, -1, :]

    f_out, i_out = pl.pallas_call(
        _tc_body,
        out_shape=[
            jax.ShapeDtypeStruct((B, 24), jnp.float32),
            jax.ShapeDtypeStruct((B, 16), jnp.int32),
        ],
    )(coord_t, outputs_class, outputs_centerness, tgt)

    traj_v = jnp.transpose(outputs_traj, (0, 2, 3, 1))

    rows, tl7 = pl.pallas_call(
        _gather_body,
        grid_spec=pltpu.PrefetchScalarGridSpec(
            num_scalar_prefetch=1,
            grid=(B,),
            in_specs=[
                pl.BlockSpec(
                    (1, T, 2, 128),
                    lambda b, idx_ref, s=s: (b, 0, 0, idx_ref[b, s] // 128))
                for s in range(7)
            ] + [
                pl.BlockSpec((1, T, 2), lambda b, idx_ref: (b, 0, 0)),
            ],
            out_specs=[
                pl.BlockSpec((1, 7, T, 2), lambda b, idx_ref: (b, 0, 0, 0)),
                pl.BlockSpec((1, 1, 1, 1), lambda b, idx_ref: (b, 0, 0, 0)),
            ],
        ),
        out_shape=[
            jax.ShapeDtypeStruct((B, 7, T, 2), jnp.float32),
            jax.ShapeDtypeStruct((B, 1, 1, 1), jnp.float32),
        ],
    )(i_out, *([traj_v] * 7), gt_points)

    total_loss = f_out[:, 0] + tl7[:, 0, 0, 0]
    de = f_out[:, 1]
    pred_probs = f_out[:, 2:8]
    pred_goals = jnp.stack([f_out[:, 8:14], f_out[:, 14:20]], axis=-1)
    pred_trajs = rows[:, 1:7]
    return (total_loss, de, pred_goals, pred_probs, pred_trajs)


# speedup vs baseline: 16.3558x; 1.0467x over previous
import functools

import jax
import jax.numpy as jnp
from jax import lax
from jax.experimental import pallas as pl
from jax.experimental.pallas import tpu as pltpu
from jax.experimental.pallas import tpu_sc as plsc

B = 16
N = 20000
NP = 20480
T = 30
EVAL_NUM = 6
NMS_THRESHOLD = 2.0
_BIG_I = 2 ** 30
_EPS = 1e-6


def _smooth_l1_elt(d):
    ad = jnp.abs(d)
    return jnp.where(ad < 1.0, 0.5 * d * d, ad - 0.5)


def _tc_body(co_ref, cls_ref, cen_ref, tgt_ref, f_ref, i_ref):
    cx = co_ref[:, 0, :]
    cy = co_ref[:, 1, :]
    cls = cls_ref[:]
    cen = cen_ref[:]
    tx = tgt_ref[:, 0:1]
    ty = tgt_ref[:, 1:2]
    lanes = lax.broadcasted_iota(jnp.int32, (B, N), 1)

    dx = cx - tx
    dy = cy - ty
    dist = jnp.sqrt(dx * dx + dy * dy + 1e-12)
    sc = cls * cen

    d_cur = dist
    cls_sum = jnp.zeros((B, 1), jnp.float32)
    idx0 = None
    pick0 = None
    dist0 = None
    for k in range(EVAL_NUM):
        m = jnp.min(d_cur, axis=1, keepdims=True)
        j = jnp.min(jnp.where(d_cur == m, lanes, _BIG_I), axis=1, keepdims=True)
        pick = lanes == j
        p = jnp.sum(jnp.where(pick, cls, 0.0), axis=1, keepdims=True)
        p = jnp.clip(p, _EPS, 1.0 - _EPS)
        cls_sum = cls_sum - jnp.log(p)
        if k == 0:
            idx0, pick0, dist0 = j, pick, m
        d_cur = jnp.where(pick, jnp.inf, d_cur)
    class_loss = cls_sum / EVAL_NUM

    px = jnp.sum(jnp.where(pick0, cx, 0.0), axis=1, keepdims=True)
    py = jnp.sum(jnp.where(pick0, cy, 0.0), axis=1, keepdims=True)
    point_loss = 0.5 * (_smooth_l1_elt(px - tx) + _smooth_l1_elt(py - ty))
    cen0 = jnp.sum(jnp.where(pick0, cen, 0.0), axis=1, keepdims=True)
    cgt = jnp.where(dist0 >= 2.0, 0.0, 1.0 - jnp.sqrt(dist0 / 2.0))
    pc = jnp.clip(cen0, _EPS, 1.0 - _EPS)
    centerness_loss = -(cgt * jnp.log(pc) + (1.0 - cgt) * jnp.log(1.0 - pc))
    part_loss = class_loss + point_loss + centerness_loss

    ms = jnp.max(sc, axis=1, keepdims=True)
    bj = jnp.min(jnp.where(sc == ms, lanes, _BIG_I), axis=1, keepdims=True)
    de = jnp.sum(jnp.where(lanes == bj, dist, 0.0), axis=1, keepdims=True)

    sc_cur = sc
    probs = []
    gxs = []
    gys = []
    kidx = []
    for _ in range(EVAL_NUM):
        m = jnp.max(sc_cur, axis=1, keepdims=True)
        j = jnp.min(jnp.where(sc_cur == m, lanes, _BIG_I), axis=1, keepdims=True)
        pick = lanes == j
        cxj = jnp.sum(jnp.where(pick, cx, 0.0), axis=1, keepdims=True)
        cyj = jnp.sum(jnp.where(pick, cy, 0.0), axis=1, keepdims=True)
        probs.append(m)
        gxs.append(cxj)
        gys.append(cyj)
        kidx.append(j)
        ddx = cx - cxj
        ddy = cy - cyj
        dd2 = ddx * ddx + ddy * ddy
        sc_cur = jnp.where(dd2 < NMS_THRESHOLD * NMS_THRESHOLD, -jnp.inf,
                           sc_cur)

    zero = jnp.zeros((B, 1), jnp.float32)
    f_ref[:] = jnp.concatenate(
        [part_loss, de] + probs + gxs + gys + [zero, zero, zero, zero], axis=1)

    izero = jnp.zeros((B, 1), jnp.int32)
    i_ref[:] = jnp.concatenate([idx0] + kidx + [izero] * 9, axis=1)


def _gather_body(idx_ref, *refs):
    traj_refs = refs[:7]
    gt_ref = refs[7]
    rows_ref = refs[8]
    tl_ref = refs[9]
    b = pl.program_id(0)
    lane = lax.broadcasted_iota(jnp.int32, (1, T, 2, 128), 3)
    for s in range(7):
        off = lax.rem(idx_ref[b, s], 128)
        blk = traj_refs[s][...]
        row = jnp.sum(jnp.where(lane == off, blk, 0.0), axis=3)
        rows_ref[0, s] = row[0]
        if s == 0:
            d = row - gt_ref[...]
            sl = jnp.sum(_smooth_l1_elt(d)) * (1.0 / (2 * T))
            tl_ref[...] = jnp.zeros((1, 1, 1, 1), jnp.float32) + sl


def kernel(outputs_coord, outputs_class, outputs_traj, outputs_centerness,
           gt_points):
    coord_t = jnp.transpose(outputs_coord, (0, 2, 1))
    tgt = gt_points[:, -1, :]

    f_out, i_out = pl.pallas_call(
        _tc_body,
        out_shape=[
            jax.ShapeDtypeStruct((B, 24), jnp.float32),
            jax.ShapeDtypeStruct((B, 16), jnp.int32),
        ],
    )(coord_t, outputs_class, outputs_centerness, tgt)

    traj_v = jnp.transpose(outputs_traj, (0, 2, 3, 1))

    rows, tl7 = pl.pallas_call(
        _gather_body,
        grid_spec=pltpu.PrefetchScalarGridSpec(
            num_scalar_prefetch=1,
            grid=(B,),
            in_specs=[
                pl.BlockSpec(
                    (1, T, 2, 128),
                    lambda b, idx_ref, s=s: (b, 0, 0, idx_ref[b, s] // 128))
                for s in range(7)
            ] + [
                pl.BlockSpec((1, T, 2), lambda b, idx_ref: (b, 0, 0)),
            ],
            out_specs=[
                pl.BlockSpec((1, 7, T, 2), lambda b, idx_ref: (b, 0, 0, 0)),
                pl.BlockSpec((1, 1, 1, 1), lambda b, idx_ref: (b, 0, 0, 0)),
            ],
        ),
        out_shape=[
            jax.ShapeDtypeStruct((B, 7, T, 2), jnp.float32),
            jax.ShapeDtypeStruct((B, 1, 1, 1), jnp.float32),
        ],
    )(i_out, *([traj_v] * 7), gt_points)

    total_loss = f_out[:, 0] + tl7[:, 0, 0, 0]
    de = f_out[:, 1]
    pred_probs = f_out[:, 2:8]
    pred_goals = jnp.stack([f_out[:, 8:14], f_out[:, 14:20]], axis=-1)
    pred_trajs = rows[:, 1:7]
    return (total_loss, de, pred_goals, pred_probs, pred_trajs)
